# baseline (speedup 1.0000x reference)
import functools

import jax
import jax.numpy as jnp
from jax import lax
from jax.experimental import pallas as pl
from jax.experimental.pallas import tpu as pltpu
from jax.experimental.pallas import tpu_sc as plsc

_NC = 2
_NS = 16
_LANES = 16
_NW = _NC * _NS

_SC_FRAC_NUM = 1
_SC_FRAC_DEN = 2

_BB = 1024


def _gcd(a, b):
    while b:
        a, b = b, a % b
    return a


def _sc_pool_kernel(L, V, D, b_per_w, ctx_ref, table_ref, out_ref,
                    ctx_v, table_v, pooled_v):
    wid = lax.axis_index("s") * _NC + lax.axis_index("c")
    nchunks = D // _LANES

    pltpu.sync_copy(ctx_ref.at[pl.ds(wid * (b_per_w * L), b_per_w * L)],
                    ctx_v)
    pltpu.sync_copy(table_ref, table_v)

    zeros = jnp.zeros((_LANES,), jnp.float32)
    for c in range(nchunks):
        table_v[pl.ds(c * _LANES, _LANES)] = zeros

    inv_l = jnp.float32(1.0 / L)

    blk = _LANES // _gcd(L, _LANES)
    nvec = blk * L // _LANES

    def block_body(q, _):
        w0 = q * (blk * L)
        ivecs = [ctx_v[pl.ds(w0 + k * _LANES, _LANES)] for k in range(nvec)]
        for j in range(blk):
            acc = [zeros] * nchunks
            for l in range(L):
                w = j * L + l
                r = ivecs[w // _LANES][w % _LANES]
                rb = r * D
                for c in range(nchunks):
                    acc[c] = acc[c] + table_v[pl.ds(rb + c * _LANES, _LANES)]
            sb = (q * blk + j) * D
            for c in range(nchunks):
                pooled_v[pl.ds(sb + c * _LANES, _LANES)] = acc[c] * inv_l
        return _

    lax.fori_loop(0, b_per_w // blk, block_body, None)

    pltpu.sync_copy(pooled_v, out_ref.at[pl.ds(wid * (b_per_w * D),
                                               b_per_w * D)])


def _sc_pool(ctx_flat, table_flat, Bh, L, V, D):
    b_per_w = Bh // _NW
    mesh = plsc.VectorSubcoreMesh(core_axis_name="c", subcore_axis_name="s",
                                  num_cores=_NC)
    body = functools.partial(_sc_pool_kernel, L, V, D, b_per_w)
    return pl.kernel(
        body,
        out_type=jax.ShapeDtypeStruct((Bh * D,), jnp.float32),
        mesh=mesh,
        scratch_types=[
            pltpu.VMEM((b_per_w * L,), jnp.int32),
            pltpu.VMEM((V * D,), jnp.float32),
            pltpu.VMEM((b_per_w * D,), jnp.float32),
        ],
        compiler_params=pltpu.CompilerParams(needs_layout_passes=False),
    )(ctx_flat, table_flat)


def _tc_both_kernel(L, V, hs, inv_l, pooled_ref, ctx_ref, table_ref, w_ref,
                    b_ref, out_ref):
    i = pl.program_id(0)

    @pl.when(i < hs)
    def _proj():
        out_ref[...] = lax.dot_general(
            pooled_ref[...], w_ref[...], (((1,), (1,)), ((), ())),
            preferred_element_type=jnp.float32) + b_ref[...]

    @pl.when(i >= hs)
    def _counts():
        tz = table_ref[...]
        row = lax.broadcasted_iota(jnp.int32, tz.shape, 0)
        tz = jnp.where(row == 0, 0.0, tz)
        m = lax.dot_general(tz, w_ref[...], (((1,), (1,)), ((), ())),
                            preferred_element_type=jnp.float32)
        ctx = ctx_ref[...]
        vocab = lax.broadcasted_iota(jnp.int32, (1, V), 1)
        cnt = jnp.zeros((ctx.shape[0], V), jnp.float32)
        for l in range(L):
            cnt = cnt + (ctx[:, l:l + 1] == vocab).astype(jnp.float32)
        out_ref[...] = lax.dot_general(
            cnt, m, (((1,), (0,)), ((), ())),
            preferred_element_type=jnp.float32) * inv_l + b_ref[...]


def _tc_both(pooled, ctx2, table, W, b2d, B, Bh, L, V, D):
    hs = Bh // _BB
    n = B // _BB
    body = functools.partial(_tc_both_kernel, L, V, hs, float(1.0 / L))
    return pl.pallas_call(
        body,
        grid=(n,),
        in_specs=[
            pl.BlockSpec((_BB, D), lambda i: (jnp.minimum(i, hs - 1), 0)),
            pl.BlockSpec((_BB, L), lambda i: (jnp.maximum(i - hs, 0), 0)),
            pl.BlockSpec((V, D), lambda i: (0, 0)),
            pl.BlockSpec((V, D), lambda i: (0, 0)),
            pl.BlockSpec((1, V), lambda i: (0, 0)),
        ],
        out_specs=pl.BlockSpec((_BB, V), lambda i: (i, 0)),
        out_shape=jax.ShapeDtypeStruct((B, V), jnp.float32),
    )(pooled, ctx2, table, W, b2d)


@jax.jit
def kernel(context, table, W, b):
    B, L = context.shape
    V, D = table.shape
    Bh = (B * _SC_FRAC_NUM // _SC_FRAC_DEN) // _BB * _BB
    b2d = b.reshape(1, V)
    ctx_flat = context.reshape(-1)
    pooled = _sc_pool(lax.slice(ctx_flat, (0,), (Bh * L,)),
                      table.reshape(-1), Bh, L, V, D)
    ctx2 = lax.slice(context, (Bh, 0), (B, L))
    return _tc_both(pooled.reshape(Bh, D), ctx2, table, W, b2d,
                    B, Bh, L, V, D)


# speedup vs baseline: 1.1386x; 1.1386x over previous
import functools

import jax
import jax.numpy as jnp
from jax import lax
from jax.experimental import pallas as pl
from jax.experimental.pallas import tpu as pltpu
from jax.experimental.pallas import tpu_sc as plsc

_NC = 2
_NS = 16
_LANES = 16
_NW = _NC * _NS

_SC_FRAC_NUM = 1
_SC_FRAC_DEN = 2

_BB = 1024


def _gcd(a, b):
    while b:
        a, b = b, a % b
    return a


def _sc_pool_kernel(L, V, D, b_per_w, ctx_ref, table_ref, out_ref,
                    ctx_v, table_v, pooled_v):
    wid = lax.axis_index("s") * _NC + lax.axis_index("c")
    nchunks = D // _LANES

    pltpu.sync_copy(ctx_ref.at[pl.ds(wid * (b_per_w * L), b_per_w * L)],
                    ctx_v)
    pltpu.sync_copy(table_ref, table_v)

    zeros = jnp.zeros((_LANES,), jnp.float32)
    for c in range(nchunks):
        table_v[pl.ds(c * _LANES, _LANES)] = zeros

    inv_l = jnp.float32(1.0 / L)

    blk = _LANES // _gcd(L, _LANES)
    nvec = blk * L // _LANES

    def block_body(q, _):
        w0 = q * (blk * L)
        ivecs = [ctx_v[pl.ds(w0 + k * _LANES, _LANES)] for k in range(nvec)]
        for j in range(blk):
            acc = [zeros] * nchunks
            for l in range(L):
                w = j * L + l
                r = ivecs[w // _LANES][w % _LANES]
                rb = r * D
                for c in range(nchunks):
                    acc[c] = acc[c] + table_v[pl.ds(rb + c * _LANES, _LANES)]
            sb = (q * blk + j) * D
            for c in range(nchunks):
                pooled_v[pl.ds(sb + c * _LANES, _LANES)] = acc[c] * inv_l
        return _

    lax.fori_loop(0, b_per_w // blk, block_body, None)

    pltpu.sync_copy(pooled_v, out_ref.at[pl.ds(wid * (b_per_w * D),
                                               b_per_w * D)])


def _sc_pool(ctx_flat, table_flat, Bh, L, V, D):
    b_per_w = Bh // _NW
    mesh = plsc.VectorSubcoreMesh(core_axis_name="c", subcore_axis_name="s",
                                  num_cores=_NC)
    body = functools.partial(_sc_pool_kernel, L, V, D, b_per_w)
    return pl.kernel(
        body,
        out_type=jax.ShapeDtypeStruct((Bh * D,), jnp.float32),
        mesh=mesh,
        scratch_types=[
            pltpu.VMEM((b_per_w * L,), jnp.int32),
            pltpu.VMEM((V * D,), jnp.float32),
            pltpu.VMEM((b_per_w * D,), jnp.float32),
        ],
        compiler_params=pltpu.CompilerParams(needs_layout_passes=False),
    )(ctx_flat, table_flat)


def _tc_counts_kernel(L, V, inv_l, ctx_ref, table_ref, w_ref, b_ref,
                      out_ref):
    tz = table_ref[...]
    row = lax.broadcasted_iota(jnp.int32, tz.shape, 0)
    tz = jnp.where(row == 0, 0.0, tz)
    m = lax.dot_general(tz, w_ref[...], (((1,), (1,)), ((), ())),
                        preferred_element_type=jnp.float32)
    ctx = ctx_ref[...]
    vocab = lax.broadcasted_iota(jnp.int32, (1, V), 1)
    cnt = jnp.zeros((ctx.shape[0], V), jnp.float32)
    for l in range(L):
        cnt = cnt + (ctx[:, l:l + 1] == vocab).astype(jnp.float32)
    out_ref[...] = lax.dot_general(
        cnt, m, (((1,), (0,)), ((), ())),
        preferred_element_type=jnp.float32) * inv_l + b_ref[...]


def _tc_counts(ctx2, table, W, b2d, B, Bh, L, V, D):
    n2 = (B - Bh) // _BB
    hs = Bh // _BB
    body = functools.partial(_tc_counts_kernel, L, V, float(1.0 / L))
    return pl.pallas_call(
        body,
        grid=(n2,),
        in_specs=[
            pl.BlockSpec((_BB, L), lambda i: (i, 0)),
            pl.BlockSpec((V, D), lambda i: (0, 0)),
            pl.BlockSpec((V, D), lambda i: (0, 0)),
            pl.BlockSpec((1, V), lambda i: (0, 0)),
        ],
        out_specs=pl.BlockSpec((_BB, V), lambda i: (hs + i, 0)),
        out_shape=jax.ShapeDtypeStruct((B, V), jnp.float32),
    )(ctx2, table, W, b2d)


def _tc_proj_kernel(x_ref, w_ref, b_ref, dummy_ref, out_ref):
    del dummy_ref
    out_ref[...] = lax.dot_general(
        x_ref[...], w_ref[...],
        (((1,), (1,)), ((), ())),
        preferred_element_type=jnp.float32,
    ) + b_ref[...]


def _tc_proj(pooled, W, b2d, partial_out, B, Bh, V, D):
    hs = Bh // _BB
    return pl.pallas_call(
        _tc_proj_kernel,
        grid=(hs,),
        in_specs=[
            pl.BlockSpec((_BB, D), lambda i: (i, 0)),
            pl.BlockSpec((V, D), lambda i: (0, 0)),
            pl.BlockSpec((1, V), lambda i: (0, 0)),
            pl.BlockSpec((8, 128), lambda i: (0, 0)),
        ],
        out_specs=pl.BlockSpec((_BB, V), lambda i: (i, 0)),
        out_shape=jax.ShapeDtypeStruct((B, V), jnp.float32),
        input_output_aliases={3: 0},
    )(pooled, W, b2d, partial_out)


@jax.jit
def kernel(context, table, W, b):
    B, L = context.shape
    V, D = table.shape
    Bh = (B * _SC_FRAC_NUM // _SC_FRAC_DEN) // _BB * _BB
    b2d = b.reshape(1, V)
    ctx_flat = context.reshape(-1)
    pooled = _sc_pool(lax.slice(ctx_flat, (0,), (Bh * L,)),
                      table.reshape(-1), Bh, L, V, D)
    partial_out = _tc_counts(lax.slice(context, (Bh, 0), (B, L)),
                             table, W, b2d, B, Bh, L, V, D)
    return _tc_proj(pooled.reshape(Bh, D), W, b2d, partial_out, B, Bh, V, D)


# speedup vs baseline: 1.1886x; 1.0440x over previous
import functools

import jax
import jax.numpy as jnp
from jax import lax
from jax.experimental import pallas as pl
from jax.experimental.pallas import tpu as pltpu
from jax.experimental.pallas import tpu_sc as plsc

_NC = 2
_NS = 16
_LANES = 16
_NW = _NC * _NS

_SC_FRAC_NUM = 5
_SC_FRAC_DEN = 8

_BB = 1024


def _gcd(a, b):
    while b:
        a, b = b, a % b
    return a


def _sc_pool_kernel(L, V, D, b_per_w, ctx_ref, table_ref, out_ref,
                    ctx_v, table_v, pooled_v):
    wid = lax.axis_index("s") * _NC + lax.axis_index("c")
    nchunks = D // _LANES

    pltpu.sync_copy(ctx_ref.at[pl.ds(wid * (b_per_w * L), b_per_w * L)],
                    ctx_v)
    pltpu.sync_copy(table_ref, table_v)

    zeros = jnp.zeros((_LANES,), jnp.float32)
    for c in range(nchunks):
        table_v[pl.ds(c * _LANES, _LANES)] = zeros

    inv_l = jnp.float32(1.0 / L)

    blk = _LANES // _gcd(L, _LANES)
    nvec = blk * L // _LANES

    def block_body(q, _):
        w0 = q * (blk * L)
        ivecs = [ctx_v[pl.ds(w0 + k * _LANES, _LANES)] for k in range(nvec)]
        for j in range(blk):
            acc = [zeros] * nchunks
            for l in range(L):
                w = j * L + l
                r = ivecs[w // _LANES][w % _LANES]
                rb = r * D
                for c in range(nchunks):
                    acc[c] = acc[c] + table_v[pl.ds(rb + c * _LANES, _LANES)]
            sb = (q * blk + j) * D
            for c in range(nchunks):
                pooled_v[pl.ds(sb + c * _LANES, _LANES)] = acc[c] * inv_l
        return _

    lax.fori_loop(0, b_per_w // blk, block_body, None)

    pltpu.sync_copy(pooled_v, out_ref.at[pl.ds(wid * (b_per_w * D),
                                               b_per_w * D)])


def _sc_pool(ctx_flat, table_flat, Bh, L, V, D):
    b_per_w = Bh // _NW
    mesh = plsc.VectorSubcoreMesh(core_axis_name="c", subcore_axis_name="s",
                                  num_cores=_NC)
    body = functools.partial(_sc_pool_kernel, L, V, D, b_per_w)
    return pl.kernel(
        body,
        out_type=jax.ShapeDtypeStruct((Bh * D,), jnp.float32),
        mesh=mesh,
        scratch_types=[
            pltpu.VMEM((b_per_w * L,), jnp.int32),
            pltpu.VMEM((V * D,), jnp.float32),
            pltpu.VMEM((b_per_w * D,), jnp.float32),
        ],
        compiler_params=pltpu.CompilerParams(needs_layout_passes=False),
    )(ctx_flat, table_flat)


def _tc_counts_kernel(L, V, inv_l, ctx_ref, table_ref, w_ref, b_ref,
                      out_ref):
    tz = table_ref[...]
    row = lax.broadcasted_iota(jnp.int32, tz.shape, 0)
    tz = jnp.where(row == 0, 0.0, tz)
    m = lax.dot_general(tz, w_ref[...], (((1,), (1,)), ((), ())),
                        preferred_element_type=jnp.float32)
    ctx = ctx_ref[...]
    vocab = lax.broadcasted_iota(jnp.int32, (1, V), 1)
    cnt = jnp.zeros((ctx.shape[0], V), jnp.float32)
    for l in range(L):
        cnt = cnt + (ctx[:, l:l + 1] == vocab).astype(jnp.float32)
    out_ref[...] = lax.dot_general(
        cnt, m, (((1,), (0,)), ((), ())),
        preferred_element_type=jnp.float32) * inv_l + b_ref[...]


def _tc_counts(ctx2, table, W, b2d, B, Bh, L, V, D):
    n2 = (B - Bh) // _BB
    hs = Bh // _BB
    body = functools.partial(_tc_counts_kernel, L, V, float(1.0 / L))
    return pl.pallas_call(
        body,
        grid=(n2,),
        in_specs=[
            pl.BlockSpec((_BB, L), lambda i: (i, 0)),
            pl.BlockSpec((V, D), lambda i: (0, 0)),
            pl.BlockSpec((V, D), lambda i: (0, 0)),
            pl.BlockSpec((1, V), lambda i: (0, 0)),
        ],
        out_specs=pl.BlockSpec((_BB, V), lambda i: (hs + i, 0)),
        out_shape=jax.ShapeDtypeStruct((B, V), jnp.float32),
    )(ctx2, table, W, b2d)


def _tc_proj_kernel(x_ref, w_ref, b_ref, dummy_ref, out_ref):
    del dummy_ref
    out_ref[...] = lax.dot_general(
        x_ref[...], w_ref[...],
        (((1,), (1,)), ((), ())),
        preferred_element_type=jnp.float32,
    ) + b_ref[...]


def _tc_proj(pooled, W, b2d, partial_out, B, Bh, V, D):
    hs = Bh // _BB
    return pl.pallas_call(
        _tc_proj_kernel,
        grid=(hs,),
        in_specs=[
            pl.BlockSpec((_BB, D), lambda i: (i, 0)),
            pl.BlockSpec((V, D), lambda i: (0, 0)),
            pl.BlockSpec((1, V), lambda i: (0, 0)),
            pl.BlockSpec((8, 128), lambda i: (0, 0)),
        ],
        out_specs=pl.BlockSpec((_BB, V), lambda i: (i, 0)),
        out_shape=jax.ShapeDtypeStruct((B, V), jnp.float32),
        input_output_aliases={3: 0},
    )(pooled, W, b2d, partial_out)


@jax.jit
def kernel(context, table, W, b):
    B, L = context.shape
    V, D = table.shape
    Bh = (B * _SC_FRAC_NUM // _SC_FRAC_DEN) // _BB * _BB
    b2d = b.reshape(1, V)
    ctx_flat = context.reshape(-1)
    pooled = _sc_pool(lax.slice(ctx_flat, (0,), (Bh * L,)),
                      table.reshape(-1), Bh, L, V, D)
    partial_out = _tc_counts(lax.slice(context, (Bh, 0), (B, L)),
                             table, W, b2d, B, Bh, L, V, D)
    return _tc_proj(pooled.reshape(Bh, D), W, b2d, partial_out, B, Bh, V, D)
